# baseline (device time: 13038 ns/iter reference)
import jax
import jax.numpy as jnp
from jax import lax
from jax.experimental import pallas as pl
from jax.experimental.pallas import tpu as pltpu

N_DEV = 8
OFFSETS = [7, 5, 6, 3, 4, 2, 1]


def kernel(x, w_mat):
    m_per, k = x.shape
    n = w_mat.shape[1]
    n_per = n // N_DEV
    m_total = N_DEV * m_per

    def gelu(y):
        c = 0.7978845608028654
        return 0.5 * y * (1.0 + jnp.tanh(c * (y + 0.044715 * y * y * y)))

    def body(x_ref, w_hbm, out_ref, w_vmem, send_buf, recv_buf,
             w_sems, send_sems, recv_sems):
        my = lax.axis_index("i")

        w_dmas = []
        for slot, d in enumerate(OFFSETS + [0]):
            t = my ^ d
            cp = pltpu.make_async_copy(
                w_hbm.at[:, pl.ds(t * n_per, n_per)],
                w_vmem.at[slot],
                w_sems.at[slot])
            cp.start()
            w_dmas.append(cp)

        barrier = pltpu.get_barrier_semaphore()
        for j in range(N_DEV):
            @pl.when(my != j)
            def _():
                pl.semaphore_signal(
                    barrier, inc=1, device_id=(j,),
                    device_id_type=pl.DeviceIdType.MESH)

        x_val = x_ref[:, :]
        for step, d in enumerate(OFFSETS):
            t = my ^ d
            w_dmas[step].wait()
            y = gelu(jnp.dot(x_val, w_vmem[step, :, :],
                             preferred_element_type=jnp.float32))
            send_buf[step, :, :] = y.astype(jnp.bfloat16)
            if step == 0:
                pl.semaphore_wait(barrier, N_DEV - 1)
            pltpu.make_async_remote_copy(
                src_ref=send_buf.at[step],
                dst_ref=recv_buf.at[step],
                send_sem=send_sems.at[step],
                recv_sem=recv_sems.at[step],
                device_id=(t,),
                device_id_type=pl.DeviceIdType.MESH,
            ).start()

        w_dmas[7].wait()
        y_own = gelu(jnp.dot(x_val, w_vmem[7, :, :],
                             preferred_element_type=jnp.float32))
        out_ref[pl.ds(my * m_per, m_per), :] = y_own

        for step, d in enumerate(OFFSETS):
            s = my ^ d
            pltpu.make_async_remote_copy(
                src_ref=send_buf.at[step],
                dst_ref=recv_buf.at[step],
                send_sem=send_sems.at[step],
                recv_sem=recv_sems.at[step],
                device_id=(s,),
                device_id_type=pl.DeviceIdType.MESH,
            ).wait_recv()
            out_ref[pl.ds(s * m_per, m_per), :] = (
                recv_buf[step, :, :].astype(jnp.float32))

        for step, d in enumerate(OFFSETS):
            t = my ^ d
            pltpu.make_async_remote_copy(
                src_ref=send_buf.at[step],
                dst_ref=recv_buf.at[step],
                send_sem=send_sems.at[step],
                recv_sem=recv_sems.at[step],
                device_id=(t,),
                device_id_type=pl.DeviceIdType.MESH,
            ).wait_send()

    return pl.pallas_call(
        body,
        out_shape=jax.ShapeDtypeStruct((m_total, n_per), jnp.float32),
        in_specs=[pl.BlockSpec(memory_space=pltpu.VMEM),
                  pl.BlockSpec(memory_space=pltpu.MemorySpace.HBM)],
        out_specs=pl.BlockSpec(memory_space=pltpu.VMEM),
        scratch_shapes=[
            pltpu.VMEM((N_DEV, k, n_per), w_mat.dtype),
            pltpu.VMEM((N_DEV - 1, m_per, n_per), jnp.bfloat16),
            pltpu.VMEM((N_DEV - 1, m_per, n_per), jnp.bfloat16),
            pltpu.SemaphoreType.DMA((N_DEV,)),
            pltpu.SemaphoreType.DMA((N_DEV - 1,)),
            pltpu.SemaphoreType.DMA((N_DEV - 1,)),
        ],
        compiler_params=pltpu.CompilerParams(collective_id=0),
    )(x, w_mat)


# device time: 12403 ns/iter; 1.0512x vs baseline; 1.0512x over previous
import jax
import jax.numpy as jnp
from jax import lax
from jax.experimental import pallas as pl
from jax.experimental.pallas import tpu as pltpu

N_DEV = 8
OFFSETS = [7, 5, 6, 3, 4, 2, 1]


def kernel(x, w_mat):
    m_per, k = x.shape
    n = w_mat.shape[1]
    n_per = n // N_DEV
    m_total = N_DEV * m_per
    k_blk = k // N_DEV

    def gelu(y):
        c = 0.7978845608028654
        return 0.5 * y * (1.0 + jnp.tanh(c * (y + 0.044715 * y * y * y)))

    def body(x_hbm, w_hbm, out_hbm, x_vmem, w_vmem, send_buf, recv_buf,
             out_stage, x_sem, w_sems, out_sems, send_sems, recv_sems):
        my = lax.axis_index("i")

        x_dma = pltpu.make_async_copy(x_hbm, x_vmem, x_sem)
        x_dma.start()
        w_dmas = []
        for r in range(N_DEV):
            cp = pltpu.make_async_copy(
                w_hbm.at[pl.ds(r * k_blk, k_blk), :],
                w_vmem.at[pl.ds(r * k_blk, k_blk), :],
                w_sems.at[r])
            cp.start()
            w_dmas.append(cp)

        barrier = pltpu.get_barrier_semaphore()
        for j in range(N_DEV):
            @pl.when(my != j)
            def _():
                pl.semaphore_signal(
                    barrier, inc=1, device_id=(j,),
                    device_id_type=pl.DeviceIdType.MESH)

        x_dma.wait()
        for r in range(N_DEV):
            w_dmas[r].wait()
        x_val = x_vmem[:, :]

        for step, d in enumerate(OFFSETS):
            t = my ^ d
            y = gelu(jnp.dot(x_val, w_vmem[:, pl.ds(t * n_per, n_per)],
                             preferred_element_type=jnp.float32))
            send_buf[step, :, :] = y.astype(jnp.bfloat16)
            if step == 0:
                pl.semaphore_wait(barrier, N_DEV - 1)
            pltpu.make_async_remote_copy(
                src_ref=send_buf.at[step],
                dst_ref=recv_buf.at[step],
                send_sem=send_sems.at[step],
                recv_sem=recv_sems.at[step],
                device_id=(t,),
                device_id_type=pl.DeviceIdType.MESH,
            ).start()

        y_own = gelu(jnp.dot(x_val, w_vmem[:, pl.ds(my * n_per, n_per)],
                             preferred_element_type=jnp.float32))
        out_stage[7, :, :] = y_own
        out_dmas = [None] * N_DEV
        out_dmas[7] = pltpu.make_async_copy(
            out_stage.at[7],
            out_hbm.at[pl.ds(my * m_per, m_per), :],
            out_sems.at[7])
        out_dmas[7].start()

        for step, d in enumerate(OFFSETS):
            s = my ^ d
            pltpu.make_async_remote_copy(
                src_ref=send_buf.at[step],
                dst_ref=recv_buf.at[step],
                send_sem=send_sems.at[step],
                recv_sem=recv_sems.at[step],
                device_id=(s,),
                device_id_type=pl.DeviceIdType.MESH,
            ).wait_recv()
            out_stage[step, :, :] = recv_buf[step, :, :].astype(jnp.float32)
            out_dmas[step] = pltpu.make_async_copy(
                out_stage.at[step],
                out_hbm.at[pl.ds(s * m_per, m_per), :],
                out_sems.at[step])
            out_dmas[step].start()

        for slot in range(N_DEV):
            out_dmas[slot].wait()
        for step, d in enumerate(OFFSETS):
            t = my ^ d
            pltpu.make_async_remote_copy(
                src_ref=send_buf.at[step],
                dst_ref=recv_buf.at[step],
                send_sem=send_sems.at[step],
                recv_sem=recv_sems.at[step],
                device_id=(t,),
                device_id_type=pl.DeviceIdType.MESH,
            ).wait_send()

    return pl.pallas_call(
        body,
        out_shape=jax.ShapeDtypeStruct((m_total, n_per), jnp.float32),
        in_specs=[pl.BlockSpec(memory_space=pl.ANY),
                  pl.BlockSpec(memory_space=pl.ANY)],
        out_specs=pl.BlockSpec(memory_space=pl.ANY),
        scratch_shapes=[
            pltpu.VMEM((m_per, k), x.dtype),
            pltpu.VMEM((k, n), w_mat.dtype),
            pltpu.VMEM((N_DEV - 1, m_per, n_per), jnp.bfloat16),
            pltpu.VMEM((N_DEV - 1, m_per, n_per), jnp.bfloat16),
            pltpu.VMEM((N_DEV, m_per, n_per), jnp.float32),
            pltpu.SemaphoreType.DMA,
            pltpu.SemaphoreType.DMA((N_DEV,)),
            pltpu.SemaphoreType.DMA((N_DEV,)),
            pltpu.SemaphoreType.DMA((N_DEV - 1,)),
            pltpu.SemaphoreType.DMA((N_DEV - 1,)),
        ],
        compiler_params=pltpu.CompilerParams(
            collective_id=0,
            vmem_limit_bytes=128 * 1024 * 1024,
        ),
    )(x, w_mat)
